# baseline (speedup 1.0000x reference)
import functools

import jax
import jax.numpy as jnp
from jax import lax
from jax.experimental import pallas as pl
from jax.experimental.pallas import tpu as pltpu
from jax.experimental.pallas import tpu_sc as plsc

N = 10000
M = 10112
C = 128
ACT = 8
G = N // ACT
E = 320000
LANES = 128
IDXW = 128
NSC = 2
NT = 16
RPT = 80
EPAD = NSC * NT * RPT * IDXW
STRIPE = M // NT

@functools.cache
def _sc_mesh():
    return plsc.VectorSubcoreMesh(core_axis_name="c", subcore_axis_name="s",
                                  num_cores=NSC, num_subcores=NT)


def _zero_rows(zb, n_rows, n_minor):
    def body(i, _):
        for u in range(n_minor // 16):
            zb[i, pl.ds(u * 16, 16)] = jnp.zeros((16,), jnp.float32)
        return 0
    lax.fori_loop(0, n_rows, body, 0)


def _zero_stripe(zb, sh, base, n_rows):
    done = 0
    while done + n_rows <= STRIPE:
        pltpu.sync_copy(zb, sh.at[pl.ds(base + done, n_rows)])
        done += n_rows
    if done < STRIPE:
        pltpu.sync_copy(zb.at[pl.ds(0, STRIPE - done)],
                        sh.at[pl.ds(base + done, STRIPE - done)])


def _sc_deg_body(dstr_h, deg_out, idx_v, ones_v, zb, deg_sh):
    cid = lax.axis_index("c")
    sid = lax.axis_index("s")

    def fill(i, _):
        ones_v[i, :] = jnp.ones((16,), jnp.float32)
        return 0
    lax.fori_loop(0, LANES, fill, 0)
    _zero_rows(zb, LANES, 16)
    _zero_stripe(zb, deg_sh, sid * STRIPE, LANES)
    plsc.subcore_barrier()

    r0 = (cid * NT + sid) * RPT
    pltpu.sync_copy(dstr_h.at[pl.ds(r0, RPT)], idx_v)

    def step(j, _):
        pltpu.sync_copy(ones_v, deg_sh.at[idx_v.at[j]], add=True)
        return 0
    lax.fori_loop(0, RPT, step, 0)

    plsc.subcore_barrier()
    pltpu.sync_copy(deg_sh.at[pl.ds(sid * STRIPE, STRIPE)],
                    deg_out.at[cid, pl.ds(sid * STRIPE, STRIPE)])


def _sc_deg(dstr):
    run = pl.kernel(
        _sc_deg_body,
        out_type=jax.ShapeDtypeStruct((NSC, M, 16), jnp.float32),
        mesh=_sc_mesh(),
        scratch_types=[
            pltpu.VMEM((RPT, IDXW), jnp.int32),
            pltpu.VMEM((LANES, 16), jnp.float32),
            pltpu.VMEM((LANES, 16), jnp.float32),
            pltpu.VMEM_SHARED((M, 16), jnp.float32),
        ],
    )
    return run(dstr)


def _sc_gcn_body(z_h, srcr_h, dstr_h, yp_out, src_v, dst_v, buf_a,
                 sem_a, y_sh):
    cid = lax.axis_index("c")
    sid = lax.axis_index("s")

    _zero_rows(buf_a, LANES, C)
    _zero_stripe(buf_a, y_sh, sid * STRIPE, LANES)
    plsc.subcore_barrier()

    r0 = (cid * NT + sid) * RPT
    pltpu.sync_copy(srcr_h.at[pl.ds(r0, RPT)], src_v)
    pltpu.sync_copy(dstr_h.at[pl.ds(r0, RPT)], dst_v)

    def step(j, _):
        pltpu.async_copy(z_h.at[src_v.at[j]], buf_a, sem_a).wait()
        pltpu.sync_copy(buf_a, y_sh.at[dst_v.at[j]], add=True)
        return 0
    lax.fori_loop(0, RPT, step, 0)

    plsc.subcore_barrier()
    pltpu.sync_copy(y_sh.at[pl.ds(sid * STRIPE, STRIPE)],
                    yp_out.at[cid, pl.ds(sid * STRIPE, STRIPE)])


def _sc_gcn(zs, srcr, dstr):
    run = pl.kernel(
        _sc_gcn_body,
        out_type=jax.ShapeDtypeStruct((NSC, M, C), jnp.float32),
        mesh=_sc_mesh(),
        scratch_types=[
            pltpu.VMEM((RPT, IDXW), jnp.int32),
            pltpu.VMEM((RPT, IDXW), jnp.int32),
            pltpu.VMEM((LANES, C), jnp.float32),
            pltpu.SemaphoreType.DMA,
            pltpu.VMEM_SHARED((M, C), jnp.float32),
        ],
    )
    return run(zs, srcr, dstr)


def _tc_first_body(deg_ref, x_ref, w_ref, dis_ref, zs_ref):
    deg = deg_ref[0, :, 0:1] + deg_ref[1, :, 0:1] + 1.0
    iot = lax.broadcasted_iota(jnp.int32, (M, 1), 0)
    dis = jnp.where(iot < N, lax.rsqrt(deg), 0.0)
    dis_ref[...] = dis
    zs_ref[...] = dis * jnp.dot(x_ref[...], w_ref[...],
                                preferred_element_type=jnp.float32)


def _tc_first(deg2, state_p, w1):
    return pl.pallas_call(
        _tc_first_body,
        out_shape=(jax.ShapeDtypeStruct((M, 1), jnp.float32),
                   jax.ShapeDtypeStruct((M, C), jnp.float32)),
    )(deg2, state_p, w1)


def _tc_mid_body(yp_ref, zs_ref, dis_ref, b_ref, w_ref, x_out, zs_out):
    dis = dis_ref[...]
    y = yp_ref[0] + yp_ref[1] + zs_ref[...]
    x = jnp.maximum(dis * y + b_ref[...], 0.0)
    x_out[...] = x
    zs_out[...] = dis * jnp.dot(x, w_ref[...],
                                preferred_element_type=jnp.float32)


def _tc_mid(yp, zs, dis, b_prev, w_next):
    return pl.pallas_call(
        _tc_mid_body,
        out_shape=(jax.ShapeDtypeStruct((M, C), jnp.float32),
                   jax.ShapeDtypeStruct((M, C), jnp.float32)),
    )(yp, zs, dis, b_prev, w_next)


def _tc_x5_body(yp_ref, zs_ref, dis_ref, b_ref, x_out):
    y = yp_ref[0] + yp_ref[1] + zs_ref[...]
    x_out[...] = jnp.maximum(dis_ref[...] * y + b_ref[...], 0.0)


def _tc_x5(yp, zs, dis, b_prev):
    return pl.pallas_call(
        _tc_x5_body,
        out_shape=jax.ShapeDtypeStruct((M, C), jnp.float32),
    )(yp, zs, dis, b_prev)


def _tc_final_body(x1, x2, x3, x4, x5_ref,
                   state_ref, act_ref, l1wt, l2wt, l3wt, l1b, l2b, l3b,
                   out_ref, y2_sc):
    x5 = x5_ref[...]
    w = l1wt[...]
    mm = functools.partial(jnp.dot, preferred_element_type=jnp.float32)
    y = mm(x1[0:N, :], w[0:C, :])
    y += mm(x2[0:N, :], w[C:2 * C, :])
    y += mm(x3[0:N, :], w[2 * C:3 * C, :])
    y += mm(x4[0:N, :], w[3 * C:4 * C, :])
    y += mm(x5[0:N, :], w[4 * C:5 * C, :])
    y += mm(state_ref[0:N, :], w[5 * C:6 * C, :])
    y += act_ref[...] * w[6 * C:6 * C + 1, :]
    y1 = jnp.maximum(y + l1b[...], 0.0)
    y2 = jnp.maximum(mm(y1, l2wt[...]) + l2b[...], 0.0)
    y2_sc[...] = y2
    acc = y2_sc[0:G, :]
    for j in range(1, ACT):
        acc += y2_sc[j * G:(j + 1) * G, :]
    out_ref[...] = mm(acc, l3wt[...]) + l3b[...]


def _tc_final(x1, x2, x3, x4, x5, state_p, act_col,
              l1wt, l2wt, l3wt, l1b, l2b, l3b):
    return pl.pallas_call(
        _tc_final_body,
        out_shape=jax.ShapeDtypeStruct((G, 1), jnp.float32),
        scratch_shapes=[pltpu.VMEM((N, 32), jnp.float32)],
    )(x1, x2, x3, x4, x5, state_p, act_col,
      l1wt, l2wt, l3wt, l1b, l2b, l3b)


def kernel(state, edge_index, action, W1, b1, W2, b2, W3, b3,
           lin1W, lin1b, lin2W, lin2b, lin3W, lin3b):
    state_p = state.reshape(G, ACT, C).transpose(1, 0, 2).reshape(N, C)
    state_p = jnp.concatenate(
        [state_p, jnp.zeros((M - N, C), jnp.float32)], axis=0)
    src = edge_index[0]
    dst = edge_index[1]
    srcp = (src % ACT) * G + src // ACT
    dstp = (dst % ACT) * G + dst // ACT
    pad_src = jnp.full((EPAD - E,), N, jnp.int32)
    pad_dst = N + (jnp.arange(EPAD - E, dtype=jnp.int32) % (M - N))
    srcr = jnp.concatenate([srcp, pad_src]).reshape(-1, IDXW)
    dstr = jnp.concatenate([dstp, pad_dst]).reshape(-1, IDXW)
    act_col = action.T.reshape(N, 1)

    deg2 = _sc_deg(dstr)
    dis, zs = _tc_first(deg2, state_p, W1)
    xs = []
    for b_prev, w_next in ((b1, W2), (b2, W3), (b3, W3), (b3, W3)):
        yp = _sc_gcn(zs, srcr, dstr)
        x_prev, zs = _tc_mid(yp, zs, dis, b_prev.reshape(1, C), w_next)
        xs.append(x_prev)
    yp5 = _sc_gcn(zs, srcr, dstr)
    x5 = _tc_x5(yp5, zs, dis, b3.reshape(1, C))
    out = _tc_final(xs[0], xs[1], xs[2], xs[3], x5, state_p, act_col,
                    lin1W.T, lin2W.T, lin3W.T,
                    lin1b.reshape(1, -1), lin2b.reshape(1, -1),
                    lin3b.reshape(1, 1))
    return out.reshape(G)


# speedup vs baseline: 1.0768x; 1.0768x over previous
import functools

import jax
import jax.numpy as jnp
from jax import lax
from jax.experimental import pallas as pl
from jax.experimental.pallas import tpu as pltpu
from jax.experimental.pallas import tpu_sc as plsc

N = 10000
M = 10112
C = 128
ACT = 8
G = N // ACT
E = 320000
LANES = 128
NSC = 2
NT = 16
NR = 2560
K0 = 112
K1 = (NR // NT) - K0
KMAX = max(K0, K1)
NRA = NR + KMAX
DRPT = NR // (NSC * NT)
EPAD = NR * LANES
STRIPE = M // NT


@functools.cache
def _sc_mesh():
    return plsc.VectorSubcoreMesh(core_axis_name="c", subcore_axis_name="s",
                                  num_cores=NSC, num_subcores=NT)


def _zero_rows(zb, n_rows, n_minor):
    def body(i, _):
        for u in range(n_minor // 16):
            zb[i, pl.ds(u * 16, 16)] = jnp.zeros((16,), jnp.float32)
        return 0
    lax.fori_loop(0, n_rows, body, 0)


def _zero_stripe(zb, sh, base, n_rows):
    done = 0
    while done + n_rows <= STRIPE:
        pltpu.sync_copy(zb, sh.at[pl.ds(base + done, n_rows)])
        done += n_rows
    if done < STRIPE:
        pltpu.sync_copy(zb.at[pl.ds(0, STRIPE - done)],
                        sh.at[pl.ds(base + done, STRIPE - done)])


def _sc_deg_body(dstr_h, deg_out, idx_v, ones_v, zb, deg_sh):
    cid = lax.axis_index("c")
    sid = lax.axis_index("s")

    def fill(i, _):
        ones_v[i, :] = jnp.ones((16,), jnp.float32)
        return 0
    lax.fori_loop(0, LANES, fill, 0)
    _zero_rows(zb, LANES, 16)
    _zero_stripe(zb, deg_sh, sid * STRIPE, LANES)
    plsc.subcore_barrier()

    r0 = (cid * NT + sid) * DRPT
    pltpu.sync_copy(dstr_h.at[pl.ds(r0, DRPT)], idx_v)

    def step(j, _):
        pltpu.sync_copy(ones_v, deg_sh.at[idx_v.at[j]], add=True)
        return 0
    lax.fori_loop(0, DRPT, step, 0)

    plsc.subcore_barrier()
    pltpu.sync_copy(deg_sh.at[pl.ds(sid * STRIPE, STRIPE)],
                    deg_out.at[cid, pl.ds(sid * STRIPE, STRIPE)])


def _sc_deg(dstr):
    run = pl.kernel(
        _sc_deg_body,
        out_type=jax.ShapeDtypeStruct((NSC, M, 16), jnp.float32),
        mesh=_sc_mesh(),
        scratch_types=[
            pltpu.VMEM((DRPT, LANES), jnp.int32),
            pltpu.VMEM((LANES, 16), jnp.float32),
            pltpu.VMEM((LANES, 16), jnp.float32),
            pltpu.VMEM_SHARED((M, 16), jnp.float32),
        ],
    )
    return run(dstr)


def _sc_gcn_body(z_h, srcr_h, dstr_h, yp_out, src_v, dst_v, buf, sem, y_sh):
    cid = lax.axis_index("c")
    sid = lax.axis_index("s")

    _zero_rows(buf, LANES, C)
    _zero_stripe(buf, y_sh, sid * STRIPE, LANES)
    plsc.subcore_barrier()

    nrows = jnp.where(cid == 0, K0, K1)
    r0 = jnp.where(cid == 0, sid * K0, NT * K0 + sid * K1)
    pltpu.sync_copy(srcr_h.at[pl.ds(r0, KMAX)], src_v)
    pltpu.sync_copy(dstr_h.at[pl.ds(r0, KMAX)], dst_v)

    def step(j, _):
        pltpu.async_copy(z_h.at[src_v.at[j]], buf, sem).wait()
        pltpu.sync_copy(buf, y_sh.at[dst_v.at[j]], add=True)
        return 0
    lax.fori_loop(0, nrows, step, 0)

    plsc.subcore_barrier()
    pltpu.sync_copy(y_sh.at[pl.ds(sid * STRIPE, STRIPE)],
                    yp_out.at[cid, pl.ds(sid * STRIPE, STRIPE)])


def _sc_gcn(zs, srcr, dstr):
    run = pl.kernel(
        _sc_gcn_body,
        out_type=jax.ShapeDtypeStruct((NSC, M, C), jnp.float32),
        mesh=_sc_mesh(),
        scratch_types=[
            pltpu.VMEM((KMAX, LANES), jnp.int32),
            pltpu.VMEM((KMAX, LANES), jnp.int32),
            pltpu.VMEM((LANES, C), jnp.float32),
            pltpu.SemaphoreType.DMA,
            pltpu.VMEM_SHARED((M, C), jnp.float32),
        ],
    )
    return run(zs, srcr, dstr)


def _tc_first_body(deg_ref, x_ref, w_ref, dis_ref, zs_ref):
    deg = deg_ref[0, :, 0:1] + deg_ref[1, :, 0:1] + 1.0
    iot = lax.broadcasted_iota(jnp.int32, (M, 1), 0)
    dis = jnp.where(iot < N, lax.rsqrt(deg), 0.0)
    dis_ref[...] = dis
    zs_ref[...] = dis * jnp.dot(x_ref[...], w_ref[...],
                                preferred_element_type=jnp.float32)


def _tc_first(deg2, state_p, w1):
    return pl.pallas_call(
        _tc_first_body,
        out_shape=(jax.ShapeDtypeStruct((M, 1), jnp.float32),
                   jax.ShapeDtypeStruct((M, C), jnp.float32)),
    )(deg2, state_p, w1)


def _tc_mid_body(yp_ref, zs_ref, dis_ref, b_ref, w_ref, x_out, zs_out):
    dis = dis_ref[...]
    y = yp_ref[0] + yp_ref[1] + zs_ref[...]
    x = jnp.maximum(dis * y + b_ref[...], 0.0)
    x_out[...] = x
    zs_out[...] = dis * jnp.dot(x, w_ref[...],
                                preferred_element_type=jnp.float32)


def _tc_mid(yp, zs, dis, b_prev, w_next):
    return pl.pallas_call(
        _tc_mid_body,
        out_shape=(jax.ShapeDtypeStruct((M, C), jnp.float32),
                   jax.ShapeDtypeStruct((M, C), jnp.float32)),
    )(yp, zs, dis, b_prev, w_next)


def _tc_x5_body(yp_ref, zs_ref, dis_ref, b_ref, x_out):
    y = yp_ref[0] + yp_ref[1] + zs_ref[...]
    x_out[...] = jnp.maximum(dis_ref[...] * y + b_ref[...], 0.0)


def _tc_x5(yp, zs, dis, b_prev):
    return pl.pallas_call(
        _tc_x5_body,
        out_shape=jax.ShapeDtypeStruct((M, C), jnp.float32),
    )(yp, zs, dis, b_prev)


def _tc_final_body(x1, x2, x3, x4, x5_ref,
                   state_ref, act_ref, l1wt, l2wt, l3wt, l1b, l2b, l3b,
                   out_ref, y2_sc):
    x5 = x5_ref[...]
    w = l1wt[...]
    mm = functools.partial(jnp.dot, preferred_element_type=jnp.float32)
    y = mm(x1[0:N, :], w[0:C, :])
    y += mm(x2[0:N, :], w[C:2 * C, :])
    y += mm(x3[0:N, :], w[2 * C:3 * C, :])
    y += mm(x4[0:N, :], w[3 * C:4 * C, :])
    y += mm(x5[0:N, :], w[4 * C:5 * C, :])
    y += mm(state_ref[0:N, :], w[5 * C:6 * C, :])
    y += act_ref[...] * w[6 * C:6 * C + 1, :]
    y1 = jnp.maximum(y + l1b[...], 0.0)
    y2 = jnp.maximum(mm(y1, l2wt[...]) + l2b[...], 0.0)
    y2_sc[...] = y2
    acc = y2_sc[0:G, :]
    for j in range(1, ACT):
        acc += y2_sc[j * G:(j + 1) * G, :]
    out_ref[...] = mm(acc, l3wt[...]) + l3b[...]


def _tc_final(x1, x2, x3, x4, x5, state_p, act_col,
              l1wt, l2wt, l3wt, l1b, l2b, l3b):
    return pl.pallas_call(
        _tc_final_body,
        out_shape=jax.ShapeDtypeStruct((G, 1), jnp.float32),
        scratch_shapes=[pltpu.VMEM((N, 32), jnp.float32)],
    )(x1, x2, x3, x4, x5, state_p, act_col,
      l1wt, l2wt, l3wt, l1b, l2b, l3b)


def kernel(state, edge_index, action, W1, b1, W2, b2, W3, b3,
           lin1W, lin1b, lin2W, lin2b, lin3W, lin3b):
    state_p = state.reshape(G, ACT, C).transpose(1, 0, 2).reshape(N, C)
    state_p = jnp.concatenate(
        [state_p, jnp.zeros((M - N, C), jnp.float32)], axis=0)
    src = edge_index[0]
    dst = edge_index[1]
    srcp = (src % ACT) * G + src // ACT
    dstp = (dst % ACT) * G + dst // ACT
    pad_n = NRA * LANES - E
    srcr = jnp.concatenate(
        [srcp, jnp.full((pad_n,), N, jnp.int32)]).reshape(NRA, LANES)
    pad_dst = N + (jnp.arange(pad_n, dtype=jnp.int32) % (M - N))
    dstr = jnp.concatenate([dstp, pad_dst]).reshape(NRA, LANES)
    act_col = action.T.reshape(N, 1)

    deg2 = _sc_deg(dstr)
    dis, zs = _tc_first(deg2, state_p, W1)
    xs = []
    for b_prev, w_next in ((b1, W2), (b2, W3), (b3, W3), (b3, W3)):
        yp = _sc_gcn(zs, srcr, dstr)
        x_prev, zs = _tc_mid(yp, zs, dis, b_prev.reshape(1, C), w_next)
        xs.append(x_prev)
    yp5 = _sc_gcn(zs, srcr, dstr)
    x5 = _tc_x5(yp5, zs, dis, b3.reshape(1, C))
    out = _tc_final(xs[0], xs[1], xs[2], xs[3], x5, state_p, act_col,
                    lin1W.T, lin2W.T, lin3W.T,
                    lin1b.reshape(1, -1), lin2b.reshape(1, -1),
                    lin3b.reshape(1, 1))
    return out.reshape(G)


# speedup vs baseline: 1.0779x; 1.0010x over previous
import functools

import jax
import jax.numpy as jnp
from jax import lax
from jax.experimental import pallas as pl
from jax.experimental.pallas import tpu as pltpu
from jax.experimental.pallas import tpu_sc as plsc

N = 10000
M = 10112
C = 128
ACT = 8
G = N // ACT
E = 320000
LANES = 128
NSC = 2
NT = 16
NR = 2560
K0 = 112
K1 = (NR // NT) - K0
KMAX = max(K0, K1)
NRA = NR + KMAX
DRPT = NR // (NSC * NT)
EPAD = NR * LANES
STRIPE = M // NT


@functools.cache
def _sc_mesh():
    return plsc.VectorSubcoreMesh(core_axis_name="c", subcore_axis_name="s",
                                  num_cores=NSC, num_subcores=NT)


def _zero_rows(zb, n_rows, n_minor):
    def body(i, _):
        for u in range(n_minor // 16):
            zb[i, pl.ds(u * 16, 16)] = jnp.zeros((16,), jnp.float32)
        return 0
    lax.fori_loop(0, n_rows, body, 0)


def _zero_stripe(zb, sh, base, n_rows):
    done = 0
    while done + n_rows <= STRIPE:
        pltpu.sync_copy(zb, sh.at[pl.ds(base + done, n_rows)])
        done += n_rows
    if done < STRIPE:
        pltpu.sync_copy(zb.at[pl.ds(0, STRIPE - done)],
                        sh.at[pl.ds(base + done, STRIPE - done)])


def _sc_deg_body(dstr_h, deg_out, idx_v, ones_v, zb, deg_sh):
    cid = lax.axis_index("c")
    sid = lax.axis_index("s")

    def fill(i, _):
        ones_v[i, :] = jnp.ones((16,), jnp.float32)
        return 0
    lax.fori_loop(0, LANES, fill, 0)
    _zero_rows(zb, LANES, 16)
    _zero_stripe(zb, deg_sh, sid * STRIPE, LANES)
    plsc.subcore_barrier()

    r0 = (cid * NT + sid) * DRPT
    pltpu.sync_copy(dstr_h.at[pl.ds(r0, DRPT)], idx_v)

    def step(j, _):
        pltpu.sync_copy(ones_v, deg_sh.at[idx_v.at[j]], add=True)
        return 0
    lax.fori_loop(0, DRPT, step, 0)

    plsc.subcore_barrier()
    pltpu.sync_copy(deg_sh.at[pl.ds(sid * STRIPE, STRIPE)],
                    deg_out.at[cid, pl.ds(sid * STRIPE, STRIPE)])


def _sc_deg(dstr):
    run = pl.kernel(
        _sc_deg_body,
        out_type=jax.ShapeDtypeStruct((NSC, M, 16), jnp.float32),
        mesh=_sc_mesh(),
        scratch_types=[
            pltpu.VMEM((DRPT, LANES), jnp.int32),
            pltpu.VMEM((LANES, 16), jnp.float32),
            pltpu.VMEM((LANES, 16), jnp.float32),
            pltpu.VMEM_SHARED((M, 16), jnp.float32),
        ],
    )
    return run(dstr)


def _sc_gcn_body(z_h, srcr_h, dstr_h, yp_out, src_v, dst_v, buf, sem, y_sh):
    cid = lax.axis_index("c")
    sid = lax.axis_index("s")

    _zero_rows(buf, LANES, C)
    _zero_stripe(buf, y_sh, sid * STRIPE, LANES)
    plsc.subcore_barrier()

    def step(j, _):
        pltpu.async_copy(z_h.at[src_v.at[j]], buf, sem).wait()
        pltpu.sync_copy(buf, y_sh.at[dst_v.at[j]], add=True)
        return 0

    @pl.when(cid == 0)
    def _():
        r0 = sid * K0
        pltpu.sync_copy(srcr_h.at[pl.ds(r0, K0)], src_v.at[pl.ds(0, K0)])
        pltpu.sync_copy(dstr_h.at[pl.ds(r0, K0)], dst_v.at[pl.ds(0, K0)])
        lax.fori_loop(0, K0, step, 0)

    @pl.when(cid == 1)
    def _():
        r0 = NT * K0 + sid * K1
        pltpu.sync_copy(srcr_h.at[pl.ds(r0, K1)], src_v.at[pl.ds(0, K1)])
        pltpu.sync_copy(dstr_h.at[pl.ds(r0, K1)], dst_v.at[pl.ds(0, K1)])
        lax.fori_loop(0, K1, step, 0)

    plsc.subcore_barrier()
    pltpu.sync_copy(y_sh.at[pl.ds(sid * STRIPE, STRIPE)],
                    yp_out.at[cid, pl.ds(sid * STRIPE, STRIPE)])


def _sc_gcn(zs, srcr, dstr):
    run = pl.kernel(
        _sc_gcn_body,
        out_type=jax.ShapeDtypeStruct((NSC, M, C), jnp.float32),
        mesh=_sc_mesh(),
        scratch_types=[
            pltpu.VMEM((KMAX, LANES), jnp.int32),
            pltpu.VMEM((KMAX, LANES), jnp.int32),
            pltpu.VMEM((LANES, C), jnp.float32),
            pltpu.SemaphoreType.DMA,
            pltpu.VMEM_SHARED((M, C), jnp.float32),
        ],
    )
    return run(zs, srcr, dstr)


def _tc_first_body(deg_ref, x_ref, w_ref, dis_ref, zs_ref):
    deg = deg_ref[0, :, 0:1] + deg_ref[1, :, 0:1] + 1.0
    iot = lax.broadcasted_iota(jnp.int32, (M, 1), 0)
    dis = jnp.where(iot < N, lax.rsqrt(deg), 0.0)
    dis_ref[...] = dis
    zs_ref[...] = dis * jnp.dot(x_ref[...], w_ref[...],
                                preferred_element_type=jnp.float32)


def _tc_first(deg2, state_p, w1):
    return pl.pallas_call(
        _tc_first_body,
        out_shape=(jax.ShapeDtypeStruct((M, 1), jnp.float32),
                   jax.ShapeDtypeStruct((M, C), jnp.float32)),
    )(deg2, state_p, w1)


def _tc_mid_body(yp_ref, zs_ref, dis_ref, b_ref, w_ref, x_out, zs_out):
    dis = dis_ref[...]
    y = yp_ref[0] + yp_ref[1] + zs_ref[...]
    x = jnp.maximum(dis * y + b_ref[...], 0.0)
    x_out[...] = x
    zs_out[...] = dis * jnp.dot(x, w_ref[...],
                                preferred_element_type=jnp.float32)


def _tc_mid(yp, zs, dis, b_prev, w_next):
    return pl.pallas_call(
        _tc_mid_body,
        out_shape=(jax.ShapeDtypeStruct((M, C), jnp.float32),
                   jax.ShapeDtypeStruct((M, C), jnp.float32)),
    )(yp, zs, dis, b_prev, w_next)


def _tc_x5_body(yp_ref, zs_ref, dis_ref, b_ref, x_out):
    y = yp_ref[0] + yp_ref[1] + zs_ref[...]
    x_out[...] = jnp.maximum(dis_ref[...] * y + b_ref[...], 0.0)


def _tc_x5(yp, zs, dis, b_prev):
    return pl.pallas_call(
        _tc_x5_body,
        out_shape=jax.ShapeDtypeStruct((M, C), jnp.float32),
    )(yp, zs, dis, b_prev)


def _tc_final_body(x1, x2, x3, x4, x5_ref,
                   state_ref, act_ref, l1wt, l2wt, l3wt, l1b, l2b, l3b,
                   out_ref, y2_sc):
    x5 = x5_ref[...]
    w = l1wt[...]
    mm = functools.partial(jnp.dot, preferred_element_type=jnp.float32)
    y = mm(x1[0:N, :], w[0:C, :])
    y += mm(x2[0:N, :], w[C:2 * C, :])
    y += mm(x3[0:N, :], w[2 * C:3 * C, :])
    y += mm(x4[0:N, :], w[3 * C:4 * C, :])
    y += mm(x5[0:N, :], w[4 * C:5 * C, :])
    y += mm(state_ref[0:N, :], w[5 * C:6 * C, :])
    y += act_ref[...] * w[6 * C:6 * C + 1, :]
    y1 = jnp.maximum(y + l1b[...], 0.0)
    y2 = jnp.maximum(mm(y1, l2wt[...]) + l2b[...], 0.0)
    y2_sc[...] = y2
    acc = y2_sc[0:G, :]
    for j in range(1, ACT):
        acc += y2_sc[j * G:(j + 1) * G, :]
    out_ref[...] = mm(acc, l3wt[...]) + l3b[...]


def _tc_final(x1, x2, x3, x4, x5, state_p, act_col,
              l1wt, l2wt, l3wt, l1b, l2b, l3b):
    return pl.pallas_call(
        _tc_final_body,
        out_shape=jax.ShapeDtypeStruct((G, 1), jnp.float32),
        scratch_shapes=[pltpu.VMEM((N, 32), jnp.float32)],
    )(x1, x2, x3, x4, x5, state_p, act_col,
      l1wt, l2wt, l3wt, l1b, l2b, l3b)


def kernel(state, edge_index, action, W1, b1, W2, b2, W3, b3,
           lin1W, lin1b, lin2W, lin2b, lin3W, lin3b):
    state_p = state.reshape(G, ACT, C).transpose(1, 0, 2).reshape(N, C)
    state_p = jnp.concatenate(
        [state_p, jnp.zeros((M - N, C), jnp.float32)], axis=0)
    src = edge_index[0]
    dst = edge_index[1]
    srcp = (src % ACT) * G + src // ACT
    dstp = (dst % ACT) * G + dst // ACT
    pad_n = NRA * LANES - E
    srcr = jnp.concatenate(
        [srcp, jnp.full((pad_n,), N, jnp.int32)]).reshape(NRA, LANES)
    pad_dst = N + (jnp.arange(pad_n, dtype=jnp.int32) % (M - N))
    dstr = jnp.concatenate([dstp, pad_dst]).reshape(NRA, LANES)
    act_col = action.T.reshape(N, 1)

    deg2 = _sc_deg(dstr)
    dis, zs = _tc_first(deg2, state_p, W1)
    xs = []
    for b_prev, w_next in ((b1, W2), (b2, W3), (b3, W3), (b3, W3)):
        yp = _sc_gcn(zs, srcr, dstr)
        x_prev, zs = _tc_mid(yp, zs, dis, b_prev.reshape(1, C), w_next)
        xs.append(x_prev)
    yp5 = _sc_gcn(zs, srcr, dstr)
    x5 = _tc_x5(yp5, zs, dis, b3.reshape(1, C))
    out = _tc_final(xs[0], xs[1], xs[2], xs[3], x5, state_p, act_col,
                    lin1W.T, lin2W.T, lin3W.T,
                    lin1b.reshape(1, -1), lin2b.reshape(1, -1),
                    lin3b.reshape(1, 1))
    return out.reshape(G)


# speedup vs baseline: 1.1494x; 1.0664x over previous
import functools

import jax
import jax.numpy as jnp
from jax import lax
from jax.experimental import pallas as pl
from jax.experimental.pallas import tpu as pltpu
from jax.experimental.pallas import tpu_sc as plsc

N = 10000
M = 10112
C = 128
ACT = 8
G = N // ACT
E = 320000
LANES = 128
NSC = 2
NT = 16
NR = 2560
K0 = 128
K1 = (NR // NT) - K0
KMAX = max(K0, K1)
NRA = NR + KMAX
DRPT = NR // (NSC * NT)
EPAD = NR * LANES
STRIPE = M // NT


@functools.cache
def _sc_mesh():
    return plsc.VectorSubcoreMesh(core_axis_name="c", subcore_axis_name="s",
                                  num_cores=NSC, num_subcores=NT)


def _zero_rows(zb, n_rows, n_minor):
    def body(i, _):
        for u in range(n_minor // 16):
            zb[i, pl.ds(u * 16, 16)] = jnp.zeros((16,), jnp.float32)
        return 0
    lax.fori_loop(0, n_rows, body, 0)


def _zero_stripe(zb, sh, base, n_rows):
    done = 0
    while done + n_rows <= STRIPE:
        pltpu.sync_copy(zb, sh.at[pl.ds(base + done, n_rows)])
        done += n_rows
    if done < STRIPE:
        pltpu.sync_copy(zb.at[pl.ds(0, STRIPE - done)],
                        sh.at[pl.ds(base + done, STRIPE - done)])


def _sc_deg_body(dstr_h, deg_out, idx_v, ones_v, zb, deg_sh):
    cid = lax.axis_index("c")
    sid = lax.axis_index("s")

    def fill(i, _):
        ones_v[i, :] = jnp.ones((16,), jnp.float32)
        return 0
    lax.fori_loop(0, LANES, fill, 0)
    _zero_rows(zb, LANES, 16)
    _zero_stripe(zb, deg_sh, sid * STRIPE, LANES)
    plsc.subcore_barrier()

    r0 = (cid * NT + sid) * DRPT
    pltpu.sync_copy(dstr_h.at[pl.ds(r0, DRPT)], idx_v)

    def step(j, _):
        pltpu.sync_copy(ones_v, deg_sh.at[idx_v.at[j]], add=True)
        return 0
    lax.fori_loop(0, DRPT, step, 0)

    plsc.subcore_barrier()
    pltpu.sync_copy(deg_sh.at[pl.ds(sid * STRIPE, STRIPE)],
                    deg_out.at[cid, pl.ds(sid * STRIPE, STRIPE)])


def _sc_deg(dstr):
    run = pl.kernel(
        _sc_deg_body,
        out_type=jax.ShapeDtypeStruct((NSC, M, 16), jnp.float32),
        mesh=_sc_mesh(),
        scratch_types=[
            pltpu.VMEM((DRPT, LANES), jnp.int32),
            pltpu.VMEM((LANES, 16), jnp.float32),
            pltpu.VMEM((LANES, 16), jnp.float32),
            pltpu.VMEM_SHARED((M, 16), jnp.float32),
        ],
    )
    return run(dstr)


def _sc_gcn_body(z_h, srcr_h, dstr_h, yp_out, src_v, dst_v, buf, sem, y_sh):
    cid = lax.axis_index("c")
    sid = lax.axis_index("s")

    _zero_rows(buf, LANES, C)
    _zero_stripe(buf, y_sh, sid * STRIPE, LANES)
    plsc.subcore_barrier()

    def step(j, _):
        pltpu.async_copy(z_h.at[src_v.at[j]], buf, sem).wait()
        pltpu.sync_copy(buf, y_sh.at[dst_v.at[j]], add=True)
        return 0

    @pl.when(cid == 0)
    def _():
        r0 = sid * K0
        pltpu.sync_copy(srcr_h.at[pl.ds(r0, K0)], src_v.at[pl.ds(0, K0)])
        pltpu.sync_copy(dstr_h.at[pl.ds(r0, K0)], dst_v.at[pl.ds(0, K0)])
        lax.fori_loop(0, K0, step, 0)

    @pl.when(cid == 1)
    def _():
        r0 = NT * K0 + sid * K1
        pltpu.sync_copy(srcr_h.at[pl.ds(r0, K1)], src_v.at[pl.ds(0, K1)])
        pltpu.sync_copy(dstr_h.at[pl.ds(r0, K1)], dst_v.at[pl.ds(0, K1)])
        lax.fori_loop(0, K1, step, 0)

    plsc.subcore_barrier()
    pltpu.sync_copy(y_sh.at[pl.ds(sid * STRIPE, STRIPE)],
                    yp_out.at[cid, pl.ds(sid * STRIPE, STRIPE)])


def _sc_gcn(zs, srcr, dstr):
    run = pl.kernel(
        _sc_gcn_body,
        out_type=jax.ShapeDtypeStruct((NSC, M, C), jnp.float32),
        mesh=_sc_mesh(),
        scratch_types=[
            pltpu.VMEM((KMAX, LANES), jnp.int32),
            pltpu.VMEM((KMAX, LANES), jnp.int32),
            pltpu.VMEM((LANES, C), jnp.float32),
            pltpu.SemaphoreType.DMA,
            pltpu.VMEM_SHARED((M, C), jnp.float32),
        ],
    )
    return run(zs, srcr, dstr)


def _tc_first_body(deg_ref, x_ref, w_ref, dis_ref, zs_ref):
    deg = deg_ref[0, :, 0:1] + deg_ref[1, :, 0:1] + 1.0
    iot = lax.broadcasted_iota(jnp.int32, (M, 1), 0)
    dis = jnp.where(iot < N, lax.rsqrt(deg), 0.0)
    dis_ref[...] = dis
    zs_ref[...] = dis * jnp.dot(x_ref[...], w_ref[...],
                                preferred_element_type=jnp.float32)


def _tc_first(deg2, state_p, w1):
    return pl.pallas_call(
        _tc_first_body,
        out_shape=(jax.ShapeDtypeStruct((M, 1), jnp.float32),
                   jax.ShapeDtypeStruct((M, C), jnp.float32)),
    )(deg2, state_p, w1)


def _tc_mid_body(yp_ref, zs_ref, dis_ref, b_ref, w_ref, x_out, zs_out):
    dis = dis_ref[...]
    y = yp_ref[0] + yp_ref[1] + zs_ref[...]
    x = jnp.maximum(dis * y + b_ref[...], 0.0)
    x_out[...] = x
    zs_out[...] = dis * jnp.dot(x, w_ref[...],
                                preferred_element_type=jnp.float32)


def _tc_mid(yp, zs, dis, b_prev, w_next):
    return pl.pallas_call(
        _tc_mid_body,
        out_shape=(jax.ShapeDtypeStruct((M, C), jnp.float32),
                   jax.ShapeDtypeStruct((M, C), jnp.float32)),
    )(yp, zs, dis, b_prev, w_next)


def _tc_x5_body(yp_ref, zs_ref, dis_ref, b_ref, x_out):
    y = yp_ref[0] + yp_ref[1] + zs_ref[...]
    x_out[...] = jnp.maximum(dis_ref[...] * y + b_ref[...], 0.0)


def _tc_x5(yp, zs, dis, b_prev):
    return pl.pallas_call(
        _tc_x5_body,
        out_shape=jax.ShapeDtypeStruct((M, C), jnp.float32),
    )(yp, zs, dis, b_prev)


def _tc_final_body(x1, x2, x3, x4, x5_ref,
                   state_ref, act_ref, l1wt, l2wt, l3wt, l1b, l2b, l3b,
                   out_ref, y2_sc):
    x5 = x5_ref[...]
    w = l1wt[...]
    mm = functools.partial(jnp.dot, preferred_element_type=jnp.float32)
    y = mm(x1[0:N, :], w[0:C, :])
    y += mm(x2[0:N, :], w[C:2 * C, :])
    y += mm(x3[0:N, :], w[2 * C:3 * C, :])
    y += mm(x4[0:N, :], w[3 * C:4 * C, :])
    y += mm(x5[0:N, :], w[4 * C:5 * C, :])
    y += mm(state_ref[0:N, :], w[5 * C:6 * C, :])
    y += act_ref[...] * w[6 * C:6 * C + 1, :]
    y1 = jnp.maximum(y + l1b[...], 0.0)
    y2 = jnp.maximum(mm(y1, l2wt[...]) + l2b[...], 0.0)
    y2_sc[...] = y2
    acc = y2_sc[0:G, :]
    for j in range(1, ACT):
        acc += y2_sc[j * G:(j + 1) * G, :]
    out_ref[...] = mm(acc, l3wt[...]) + l3b[...]


def _tc_final(x1, x2, x3, x4, x5, state_p, act_col,
              l1wt, l2wt, l3wt, l1b, l2b, l3b):
    return pl.pallas_call(
        _tc_final_body,
        out_shape=jax.ShapeDtypeStruct((G, 1), jnp.float32),
        scratch_shapes=[pltpu.VMEM((N, 32), jnp.float32)],
    )(x1, x2, x3, x4, x5, state_p, act_col,
      l1wt, l2wt, l3wt, l1b, l2b, l3b)


def kernel(state, edge_index, action, W1, b1, W2, b2, W3, b3,
           lin1W, lin1b, lin2W, lin2b, lin3W, lin3b):
    state_p = state.reshape(G, ACT, C).transpose(1, 0, 2).reshape(N, C)
    state_p = jnp.concatenate(
        [state_p, jnp.zeros((M - N, C), jnp.float32)], axis=0)
    src = edge_index[0]
    dst = edge_index[1]
    srcp = (src % ACT) * G + src // ACT
    dstp = (dst % ACT) * G + dst // ACT
    pad_n = NRA * LANES - E
    srcr = jnp.concatenate(
        [srcp, jnp.full((pad_n,), N, jnp.int32)]).reshape(NRA, LANES)
    pad_dst = N + (jnp.arange(pad_n, dtype=jnp.int32) % (M - N))
    dstr = jnp.concatenate([dstp, pad_dst]).reshape(NRA, LANES)
    act_col = action.T.reshape(N, 1)

    deg2 = _sc_deg(dstr)
    dis, zs = _tc_first(deg2, state_p, W1)
    xs = []
    for b_prev, w_next in ((b1, W2), (b2, W3), (b3, W3), (b3, W3)):
        yp = _sc_gcn(zs, srcr, dstr)
        x_prev, zs = _tc_mid(yp, zs, dis, b_prev.reshape(1, C), w_next)
        xs.append(x_prev)
    yp5 = _sc_gcn(zs, srcr, dstr)
    x5 = _tc_x5(yp5, zs, dis, b3.reshape(1, C))
    out = _tc_final(xs[0], xs[1], xs[2], xs[3], x5, state_p, act_col,
                    lin1W.T, lin2W.T, lin3W.T,
                    lin1b.reshape(1, -1), lin2b.reshape(1, -1),
                    lin3b.reshape(1, 1))
    return out.reshape(G)
